# Initial kernel scaffold; baseline (speedup 1.0000x reference)
import jax
import jax.numpy as jnp
from jax.experimental import pallas as pl


def kernel(residue_idx_i, residue_idx_j, embed_table):
    raise NotImplementedError("write your pallas kernel here")


# speedup vs baseline: 5.4121x; 5.4121x over previous
import functools

import jax
import jax.numpy as jnp
from jax import lax
from jax.experimental import pallas as pl
from jax.experimental.pallas import tpu as pltpu
from jax.experimental.pallas import tpu_sc as plsc

NUM_RES_TYPES = 22
EMBED_DIM = 64
E = 800000
NC = 2
NS = 16
NW = NC * NS
B = 128
NCH = E // B


def _sc_body(idx_i_hbm, idx_j_hbm, ptable_hbm, out_hbm,
             idxi_v, idxj_v, cidx_v, emb_v, sem):
    wid = lax.axis_index("s") * NC + lax.axis_index("c")
    n_mine = (NCH - wid + NW - 1) // NW

    def body(t, carry):
        chunk = wid + t * NW
        base = chunk * B
        pltpu.sync_copy(idx_i_hbm.at[pl.ds(base, B)], idxi_v)
        pltpu.sync_copy(idx_j_hbm.at[pl.ds(base, B)], idxj_v)

        def cbody(r, c2):
            vi = idxi_v[pl.ds(r * 16, 16)]
            vj = idxj_v[pl.ds(r * 16, 16)]
            cidx_v[pl.ds(r * 16, 16)] = vi * NUM_RES_TYPES + vj
            return c2

        lax.fori_loop(0, B // 16, cbody, 0)
        pltpu.async_copy(ptable_hbm.at[cidx_v], emb_v, sem).wait()
        pltpu.sync_copy(emb_v, out_hbm.at[pl.ds(base, B)])
        return carry

    lax.fori_loop(0, n_mine, body, 0)


@jax.jit
def kernel(residue_idx_i, residue_idx_j, embed_table):
    ptable = jnp.concatenate(
        [jnp.repeat(embed_table, NUM_RES_TYPES, axis=0),
         jnp.tile(embed_table, (NUM_RES_TYPES, 1))], axis=1)
    mesh = plsc.VectorSubcoreMesh(core_axis_name="c", subcore_axis_name="s")
    f = functools.partial(
        pl.kernel,
        mesh=mesh,
        out_type=jax.ShapeDtypeStruct((E, 2 * EMBED_DIM), jnp.float32),
        scratch_types=[
            pltpu.VMEM((B,), jnp.int32),
            pltpu.VMEM((B,), jnp.int32),
            pltpu.VMEM((B,), jnp.int32),
            pltpu.VMEM((B, 2 * EMBED_DIM), jnp.float32),
            pltpu.SemaphoreType.DMA,
        ],
    )(_sc_body)
    return f(residue_idx_i.astype(jnp.int32), residue_idx_j.astype(jnp.int32),
             ptable)


# speedup vs baseline: 5.8955x; 1.0893x over previous
import functools

import jax
import jax.numpy as jnp
from jax import lax
from jax.experimental import pallas as pl
from jax.experimental.pallas import tpu as pltpu
from jax.experimental.pallas import tpu_sc as plsc

NUM_RES_TYPES = 22
EMBED_DIM = 64
E = 800000
NC = 2
NS = 16
NW = NC * NS
B = 256
NCH = E // B
NB = 2
IT = -(-NCH // NW)
assert IT % NB == 0


def _sc_body(idx_i_hbm, idx_j_hbm, ptable_hbm, out_hbm,
             idxi0, idxj0, cidx0, emb0, idxi1, idxj1, cidx1, emb1,
             semi0, semi1, semg0, semg1, semw0, semw1):
    wid = lax.axis_index("s") * NC + lax.axis_index("c")
    idxi = (idxi0, idxi1)
    idxj = (idxj0, idxj1)
    cidx = (cidx0, cidx1)
    emb = (emb0, emb1)
    semi = (semi0, semi1)
    semg = (semg0, semg1)
    semw = (semw0, semw1)

    def start_idx(t, b):
        ch = wid + t * NW

        @pl.when(ch < NCH)
        def _():
            base = ch * B
            pltpu.async_copy(idx_i_hbm.at[pl.ds(base, B)], idxi[b], semi[b])
            pltpu.async_copy(idx_j_hbm.at[pl.ds(base, B)], idxj[b], semi[b])

    start_idx(0, 0)
    start_idx(1, 1)

    def outer(t0, carry):
        for b in range(NB):
            t = t0 + b
            ch = wid + t * NW
            ch_prev = ch - NB * NW

            @pl.when(jnp.logical_and(t0 >= NB, ch_prev < NCH))
            def _():
                pltpu.make_async_copy(
                    emb[b], out_hbm.at[pl.ds(0, B)], semw[b]).wait()

            @pl.when(ch < NCH)
            def _():
                base = ch * B
                pltpu.make_async_copy(
                    idx_i_hbm.at[pl.ds(base, B)], idxi[b], semi[b]).wait()
                pltpu.make_async_copy(
                    idx_j_hbm.at[pl.ds(base, B)], idxj[b], semi[b]).wait()

                def cbody(r, c2):
                    vi = idxi[b][pl.ds(r * 16, 16)]
                    vj = idxj[b][pl.ds(r * 16, 16)]
                    cidx[b][pl.ds(r * 16, 16)] = vi * NUM_RES_TYPES + vj
                    return c2

                lax.fori_loop(0, B // 16, cbody, 0)
                start_idx(t + NB, b)
                pltpu.async_copy(ptable_hbm.at[cidx[b]], emb[b], semg[b]).wait()
                pltpu.async_copy(emb[b], out_hbm.at[pl.ds(base, B)], semw[b])

        return carry

    lax.fori_loop(0, IT // NB, lambda s, c: outer(s * NB, c), 0)

    for b in range(NB):
        ch_last = wid + (IT - NB + b) * NW

        @pl.when(ch_last < NCH)
        def _():
            pltpu.make_async_copy(emb[b], out_hbm.at[pl.ds(0, B)], semw[b]).wait()


@jax.jit
def kernel(residue_idx_i, residue_idx_j, embed_table):
    ptable = jnp.concatenate(
        [jnp.repeat(embed_table, NUM_RES_TYPES, axis=0),
         jnp.tile(embed_table, (NUM_RES_TYPES, 1))], axis=1)
    mesh = plsc.VectorSubcoreMesh(core_axis_name="c", subcore_axis_name="s")
    f = functools.partial(
        pl.kernel,
        mesh=mesh,
        out_type=jax.ShapeDtypeStruct((E, 2 * EMBED_DIM), jnp.float32),
        scratch_types=[
            pltpu.VMEM((B,), jnp.int32),
            pltpu.VMEM((B,), jnp.int32),
            pltpu.VMEM((B,), jnp.int32),
            pltpu.VMEM((B, 2 * EMBED_DIM), jnp.float32),
            pltpu.VMEM((B,), jnp.int32),
            pltpu.VMEM((B,), jnp.int32),
            pltpu.VMEM((B,), jnp.int32),
            pltpu.VMEM((B, 2 * EMBED_DIM), jnp.float32),
            pltpu.SemaphoreType.DMA,
            pltpu.SemaphoreType.DMA,
            pltpu.SemaphoreType.DMA,
            pltpu.SemaphoreType.DMA,
            pltpu.SemaphoreType.DMA,
            pltpu.SemaphoreType.DMA,
        ],
    )(_sc_body)
    return f(residue_idx_i.astype(jnp.int32), residue_idx_j.astype(jnp.int32),
             ptable)
